# baseline (speedup 1.0000x reference)
import functools

import jax
import jax.numpy as jnp
from jax import lax
from jax.experimental import pallas as pl
from jax.experimental.pallas import tpu as pltpu
from jax.experimental.pallas import tpu_sc as plsc

F = 26
V = 100000
D = 32
B = 16384

NC = 2
NS = 16
NW = NC * NS
ROWS = B * F
ROWS_W = ROWS // NW
CHUNK = 1024
NCHUNK = ROWS_W // CHUNK
JPC = CHUNK // 128
VPC = CHUNK // 16

VT = V // 128
VTAIL = V - VT * 128
NSLAB = F * VT
SLAB_IT = (NSLAB + NW - 1) // NW


def _sc_transpose(t_t):
    mesh = plsc.VectorSubcoreMesh(core_axis_name="c", subcore_axis_name="s")

    @functools.partial(
        pl.kernel,
        mesh=mesh,
        out_type=jax.ShapeDtypeStruct((F * V // 4, 128), jnp.float32),
        compiler_params=pltpu.CompilerParams(
            use_tc_tiling_on_sc=True, needs_layout_passes=False),
        scratch_types=[
            pltpu.VMEM((2, 32, 128), jnp.float32),
            pltpu.VMEM((2, 32, 128), jnp.float32),
            pltpu.VMEM((32, VTAIL), jnp.float32),
            pltpu.SemaphoreType.DMA,
            pltpu.SemaphoreType.DMA,
            pltpu.SemaphoreType.DMA,
            pltpu.SemaphoreType.DMA,
        ],
    )
    def k(src_hbm, dst_hbm, sbuf, dbuf, tbuf, in_s0, in_s1, out_s0, out_s1):
        w = lax.axis_index("s") * NC + lax.axis_index("c")
        lane = lax.broadcasted_iota(jnp.int32, (16,), 0)
        in_sems = (in_s0, in_s1)
        out_sems = (out_s0, out_s1)

        def slab_of(kk):
            g = w + kk * NW
            return g // VT, (g % VT) * 128

        def fire_in(kk, par):
            f, v0 = slab_of(kk)
            for e2 in range(4):
                pltpu.async_copy(
                    src_hbm.at[f, pl.ds(e2 * 8, 8), pl.ds(v0, 128)],
                    sbuf.at[par, pl.ds(e2 * 8, 8)], in_sems[par])

        def wait_in(par):
            for _ in range(4):
                pltpu.make_async_copy(
                    src_hbm.at[0, pl.ds(0, 8), pl.ds(0, 128)],
                    sbuf.at[par, pl.ds(0, 8)], in_sems[par]).wait()

        def wait_out(par):
            pltpu.make_async_copy(
                dbuf.at[par], dst_hbm.at[pl.ds(0, 32)], out_sems[par]).wait()

        def transpose_rows(src, dst, nrows):
            def trow(r, carry):
                for dv in range(4):
                    vcol = jnp.full((16,), 4 * r + dv, jnp.int32)
                    for h in range(2):
                        vals = plsc.load_gather(src, [lane + h * 16, vcol])
                        dst[r, pl.ds(dv * 32 + h * 16, 16)] = vals
                return carry

            lax.fori_loop(0, nrows, trow, 0)

        def do_slab(kk, par):
            f, v0 = slab_of(kk)
            wait_in(par)
            transpose_rows(sbuf.at[par], dbuf.at[par], 32)
            pltpu.async_copy(
                dbuf.at[par],
                dst_hbm.at[pl.ds(
                    pl.multiple_of(f * (V // 4) + v0 // 4, 32), 32)],
                out_sems[par])

        fire_in(0, 0)

        def body2(t, carry):
            for off in range(2):
                kk = 2 * t + off
                nxt = kk + 1

                @pl.when((nxt < SLAB_IT) & (w + nxt * NW < NSLAB))
                def _():
                    fire_in(nxt, 1 - off)

                @pl.when(w + kk * NW < NSLAB)
                def _():
                    @pl.when(kk >= 2)
                    def _():
                        wait_out(off)

                    do_slab(kk, off)
            return carry

        lax.fori_loop(0, (SLAB_IT + 1) // 2, body2, 0)
        wait_out(0)
        wait_out(1)

        @pl.when(w < F)
        def _():
            f = w
            for e2 in range(4):
                pltpu.sync_copy(
                    src_hbm.at[f, pl.ds(e2 * 8, 8), pl.ds(VT * 128, VTAIL)],
                    tbuf.at[pl.ds(e2 * 8, 8)])
            transpose_rows(tbuf, dbuf.at[0], VTAIL // 4)
            pltpu.sync_copy(
                dbuf.at[0, pl.ds(0, VTAIL // 4)],
                dst_hbm.at[pl.ds(
                    pl.multiple_of(f * (V // 4) + VT * 32, 8), VTAIL // 4)])

    return k(t_t)


def _sc_gather(idx2d, flat_tab):
    mesh = plsc.VectorSubcoreMesh(core_axis_name="c", subcore_axis_name="s")

    @functools.partial(
        pl.kernel,
        mesh=mesh,
        out_type=jax.ShapeDtypeStruct((ROWS, D), jnp.float32),
        compiler_params=pltpu.CompilerParams(use_tc_tiling_on_sc=False),
        scratch_types=[
            pltpu.VMEM((2, JPC, 128), jnp.int32),
            pltpu.VMEM((2, CHUNK, D), jnp.float32),
            pltpu.SemaphoreType.DMA,
            pltpu.SemaphoreType.DMA,
            pltpu.SemaphoreType.DMA,
            pltpu.SemaphoreType.DMA,
            pltpu.SemaphoreType.DMA,
            pltpu.SemaphoreType.DMA,
        ],
    )
    def k(idx_hbm, tab_hbm, out_hbm, idx_v, rows_v,
          idx_s0, idx_s1, gat_s0, gat_s1, out_s0, out_s1):
        wid = lax.axis_index("s") * NC + lax.axis_index("c")
        irow0 = wid * (ROWS_W // 128)
        orow0 = wid * ROWS_W
        lane = lax.broadcasted_iota(jnp.int32, (16,), 0)

        idx_sems = (idx_s0, idx_s1)
        gat_sems = (gat_s0, gat_s1)
        out_sems = (out_s0, out_s1)

        def start_idx(c):
            b = c & 1
            return pltpu.async_copy(
                idx_hbm.at[pl.ds(irow0 + c * JPC, JPC)], idx_v.at[b],
                idx_sems[b])

        idx_cp = {0: start_idx(0)}
        out_cp = {}
        for c in range(NCHUNK):
            b = c & 1
            if c + 1 < NCHUNK:
                idx_cp[c + 1] = start_idx(c + 1)
            idx_cp[c].wait()

            def body(v, carry):
                j = v // 8
                col = (v % 8) * 16
                pos = c * CHUNK + v * 16 + lane
                off = (pos % F) * V
                idx_v[b, j, pl.ds(col, 16)] = (
                    idx_v[b, j, pl.ds(col, 16)] + off)
                return carry

            lax.fori_loop(0, VPC, body, 0)

            if c >= 2:
                out_cp[c - 2].wait()
            gats = [
                pltpu.async_copy(
                    tab_hbm.at[idx_v.at[b, j]],
                    rows_v.at[b, pl.ds(j * 128, 128)], gat_sems[b])
                for j in range(JPC)
            ]
            for g in gats:
                g.wait()
            out_cp[c] = pltpu.async_copy(
                rows_v.at[b], out_hbm.at[pl.ds(orow0 + c * CHUNK, CHUNK)],
                out_sems[b])
        out_cp[NCHUNK - 2].wait()
        out_cp[NCHUNK - 1].wait()

    return k(idx2d, flat_tab)


def kernel(x_cat, tables):
    idx2d = x_cat.reshape(ROWS // 128, 128)
    t_t = tables.transpose(0, 2, 1)
    tab128 = _sc_transpose(t_t)
    flat_tab = tab128.reshape(F * V, D)
    out = _sc_gather(idx2d, flat_tab)
    return out.reshape(B, F * D)


# speedup vs baseline: 1.2145x; 1.2145x over previous
import functools

import jax
import jax.numpy as jnp
from jax import lax
from jax.experimental import pallas as pl
from jax.experimental.pallas import tpu as pltpu
from jax.experimental.pallas import tpu_sc as plsc

F = 26
V = 100000
D = 32
B = 16384

NC = 2
NS = 16
NW = NC * NS
ROWS = B * F
ROWS_W = ROWS // NW
CHUNK = 1024
NCHUNK = ROWS_W // CHUNK
JPC = CHUNK // 128
VPC = CHUNK // 16

VT = V // 128
VTAIL = V - VT * 128
NSLAB = F * VT
SLAB_IT = (NSLAB + NW - 1) // NW


NBUF = 4


def _sc_transpose(t_t):
    mesh = plsc.VectorSubcoreMesh(core_axis_name="c", subcore_axis_name="s")

    @functools.partial(
        pl.kernel,
        mesh=mesh,
        out_type=jax.ShapeDtypeStruct((F * V * D,), jnp.float32),
        compiler_params=pltpu.CompilerParams(
            use_tc_tiling_on_sc=True, needs_layout_passes=False),
        scratch_types=[
            pltpu.VMEM((NBUF, 32, 128), jnp.float32),
            pltpu.VMEM((4096,), jnp.float32),
            pltpu.VMEM((4096,), jnp.float32),
            pltpu.VMEM((4096,), jnp.float32),
            pltpu.VMEM((4096,), jnp.float32),
            pltpu.VMEM((32, VTAIL), jnp.float32),
            pltpu.VMEM((VTAIL * 32,), jnp.float32),
            pltpu.SemaphoreType.DMA,
            pltpu.SemaphoreType.DMA,
            pltpu.SemaphoreType.DMA,
            pltpu.SemaphoreType.DMA,
            pltpu.SemaphoreType.DMA,
            pltpu.SemaphoreType.DMA,
            pltpu.SemaphoreType.DMA,
            pltpu.SemaphoreType.DMA,
        ],
    )
    def k(src_hbm, dst_hbm, sbuf, dbuf0, dbuf1, dbuf2, dbuf3, tbuf, tdbuf,
          in_s0, in_s1, in_s2, in_s3, out_s0, out_s1, out_s2, out_s3):
        w = lax.axis_index("s") * NC + lax.axis_index("c")
        lane = lax.broadcasted_iota(jnp.int32, (16,), 0)
        dbufs = (dbuf0, dbuf1, dbuf2, dbuf3)
        in_sems = (in_s0, in_s1, in_s2, in_s3)
        out_sems = (out_s0, out_s1, out_s2, out_s3)
        patt = [
            lax.shift_right_logical(lane, 2) * 128 + (lane & 3) * 32 + kx * 512
            for kx in range(8)
        ]

        def slab_of(kk):
            g = w + kk * NW
            return g // VT, (g % VT) * 128

        def fire_in(kk, par):
            f, v0 = slab_of(kk)
            for e2 in range(4):
                pltpu.async_copy(
                    src_hbm.at[f, pl.ds(e2 * 8, 8), pl.ds(v0, 128)],
                    sbuf.at[par, pl.ds(e2 * 8, 8)], in_sems[par])

        def wait_in(par):
            for _ in range(4):
                pltpu.make_async_copy(
                    src_hbm.at[0, pl.ds(0, 8), pl.ds(0, 128)],
                    sbuf.at[par, pl.ds(0, 8)], in_sems[par]).wait()

        def wait_out(par):
            pltpu.make_async_copy(
                dbufs[par], dst_hbm.at[pl.ds(0, 4096)],
                out_sems[par]).wait()

        def do_slab(kk, par):
            f, v0 = slab_of(kk)
            wait_in(par)

            def trow(e, carry):
                esp = jnp.full((16,), e, jnp.int32)
                for kx in range(8):
                    vals = sbuf[par, e, pl.ds(kx * 16, 16)]
                    plsc.store_scatter(dbufs[par], [patt[kx] + esp], vals)
                return carry

            lax.fori_loop(0, 32, trow, 0, unroll=2)
            pltpu.async_copy(
                dbufs[par],
                dst_hbm.at[pl.ds(
                    pl.multiple_of((f * (V // 4) + v0 // 4) * 128, 1024),
                    4096)],
                out_sems[par])

        for p in range(NBUF - 1):
            fire_in(p, p)

        def body4(t, carry):
            for off in range(NBUF):
                kk = NBUF * t + off
                nxt = kk + NBUF - 1

                @pl.when((nxt < SLAB_IT) & (w + nxt * NW < NSLAB))
                def _():
                    fire_in(nxt, (off + NBUF - 1) % NBUF)

                @pl.when(w + kk * NW < NSLAB)
                def _():
                    @pl.when(kk >= NBUF)
                    def _():
                        wait_out(off)

                    do_slab(kk, off)
            return carry

        lax.fori_loop(0, (SLAB_IT + NBUF - 1) // NBUF, body4, 0)
        for p in range(NBUF):
            wait_out(p)

        @pl.when(w < F)
        def _():
            f = w
            for e2 in range(4):
                pltpu.sync_copy(
                    src_hbm.at[f, pl.ds(e2 * 8, 8), pl.ds(VT * 128, VTAIL)],
                    tbuf.at[pl.ds(e2 * 8, 8)])

            def trow(e, carry):
                esp = jnp.full((16,), e, jnp.int32)
                for kx in range(2):
                    vals = tbuf[e, pl.ds(kx * 16, 16)]
                    plsc.store_scatter(tdbuf, [patt[kx] + esp], vals)
                return carry

            lax.fori_loop(0, 32, trow, 0)
            pltpu.sync_copy(
                tdbuf,
                dst_hbm.at[pl.ds(
                    pl.multiple_of((f * (V // 4) + VT * 32) * 128, 1024),
                    VTAIL * 32)])

    return k(t_t)


def _sc_gather(idx2d, flat_tab):
    mesh = plsc.VectorSubcoreMesh(core_axis_name="c", subcore_axis_name="s")

    @functools.partial(
        pl.kernel,
        mesh=mesh,
        out_type=jax.ShapeDtypeStruct((ROWS, D), jnp.float32),
        compiler_params=pltpu.CompilerParams(use_tc_tiling_on_sc=False),
        scratch_types=[
            pltpu.VMEM((2, JPC, 128), jnp.int32),
            pltpu.VMEM((2, CHUNK, D), jnp.float32),
            pltpu.SemaphoreType.DMA,
            pltpu.SemaphoreType.DMA,
            pltpu.SemaphoreType.DMA,
            pltpu.SemaphoreType.DMA,
            pltpu.SemaphoreType.DMA,
            pltpu.SemaphoreType.DMA,
        ],
    )
    def k(idx_hbm, tab_hbm, out_hbm, idx_v, rows_v,
          idx_s0, idx_s1, gat_s0, gat_s1, out_s0, out_s1):
        wid = lax.axis_index("s") * NC + lax.axis_index("c")
        irow0 = wid * (ROWS_W // 128)
        orow0 = wid * ROWS_W
        lane = lax.broadcasted_iota(jnp.int32, (16,), 0)

        idx_sems = (idx_s0, idx_s1)
        gat_sems = (gat_s0, gat_s1)
        out_sems = (out_s0, out_s1)

        def start_idx(c):
            b = c & 1
            return pltpu.async_copy(
                idx_hbm.at[pl.ds(irow0 + c * JPC, JPC)], idx_v.at[b],
                idx_sems[b])

        idx_cp = {0: start_idx(0)}
        out_cp = {}
        for c in range(NCHUNK):
            b = c & 1
            if c + 1 < NCHUNK:
                idx_cp[c + 1] = start_idx(c + 1)
            idx_cp[c].wait()

            def body(v, carry):
                j = v // 8
                col = (v % 8) * 16
                pos = c * CHUNK + v * 16 + lane
                off = (pos % F) * V
                idx_v[b, j, pl.ds(col, 16)] = (
                    idx_v[b, j, pl.ds(col, 16)] + off)
                return carry

            lax.fori_loop(0, VPC, body, 0)

            if c >= 2:
                out_cp[c - 2].wait()
            gats = [
                pltpu.async_copy(
                    tab_hbm.at[idx_v.at[b, j]],
                    rows_v.at[b, pl.ds(j * 128, 128)], gat_sems[b])
                for j in range(JPC)
            ]
            for g in gats:
                g.wait()
            out_cp[c] = pltpu.async_copy(
                rows_v.at[b], out_hbm.at[pl.ds(orow0 + c * CHUNK, CHUNK)],
                out_sems[b])
        out_cp[NCHUNK - 2].wait()
        out_cp[NCHUNK - 1].wait()

    return k(idx2d, flat_tab)


def kernel(x_cat, tables):
    idx2d = x_cat.reshape(ROWS // 128, 128)
    t_t = tables.transpose(0, 2, 1)
    tab1d = _sc_transpose(t_t)
    flat_tab = tab1d.reshape(F * V, D)
    out = _sc_gather(idx2d, flat_tab)
    return out.reshape(B, F * D)


# speedup vs baseline: 1.2197x; 1.0042x over previous
import functools

import jax
import jax.numpy as jnp
from jax import lax
from jax.experimental import pallas as pl
from jax.experimental.pallas import tpu as pltpu
from jax.experimental.pallas import tpu_sc as plsc

F = 26
V = 100000
D = 32
B = 16384

NC = 2
NS = 16
NW = NC * NS
ROWS = B * F
ROWS_W = ROWS // NW
CHUNK = 1024
NCHUNK = ROWS_W // CHUNK
JPC = CHUNK // 128
VPC = CHUNK // 16

VT = V // 128
VTAIL = V - VT * 128
NSLAB = F * VT
SLAB_IT = (NSLAB + NW - 1) // NW


NBUF = 4


def _sc_transpose(t_t):
    mesh = plsc.VectorSubcoreMesh(core_axis_name="c", subcore_axis_name="s")

    @functools.partial(
        pl.kernel,
        mesh=mesh,
        out_type=jax.ShapeDtypeStruct((F * V * D,), jnp.float32),
        compiler_params=pltpu.CompilerParams(
            use_tc_tiling_on_sc=True, needs_layout_passes=False),
        scratch_types=[
            pltpu.VMEM((NBUF, 32, 128), jnp.float32),
            pltpu.VMEM((4096,), jnp.float32),
            pltpu.VMEM((4096,), jnp.float32),
            pltpu.VMEM((4096,), jnp.float32),
            pltpu.VMEM((4096,), jnp.float32),
            pltpu.VMEM((32, VTAIL), jnp.float32),
            pltpu.VMEM((VTAIL * 32,), jnp.float32),
            pltpu.SemaphoreType.DMA,
            pltpu.SemaphoreType.DMA,
            pltpu.SemaphoreType.DMA,
            pltpu.SemaphoreType.DMA,
            pltpu.SemaphoreType.DMA,
            pltpu.SemaphoreType.DMA,
            pltpu.SemaphoreType.DMA,
            pltpu.SemaphoreType.DMA,
        ],
    )
    def k(src_hbm, dst_hbm, sbuf, dbuf0, dbuf1, dbuf2, dbuf3, tbuf, tdbuf,
          in_s0, in_s1, in_s2, in_s3, out_s0, out_s1, out_s2, out_s3):
        w = lax.axis_index("s") * NC + lax.axis_index("c")
        lane = lax.broadcasted_iota(jnp.int32, (16,), 0)
        dbufs = (dbuf0, dbuf1, dbuf2, dbuf3)
        in_sems = (in_s0, in_s1, in_s2, in_s3)
        out_sems = (out_s0, out_s1, out_s2, out_s3)
        patt = [
            lax.shift_right_logical(lane, 2) * 128 + (lane & 3) * 32 + kx * 512
            for kx in range(8)
        ]

        def slab_of(kk):
            g = w + kk * NW
            return g // VT, (g % VT) * 128

        def fire_in(kk, par):
            f, v0 = slab_of(kk)
            for e2 in range(4):
                pltpu.async_copy(
                    src_hbm.at[f, pl.ds(e2 * 8, 8), pl.ds(v0, 128)],
                    sbuf.at[par, pl.ds(e2 * 8, 8)], in_sems[par])

        def wait_in(par):
            for _ in range(4):
                pltpu.make_async_copy(
                    src_hbm.at[0, pl.ds(0, 8), pl.ds(0, 128)],
                    sbuf.at[par, pl.ds(0, 8)], in_sems[par]).wait()

        def wait_out(par):
            pltpu.make_async_copy(
                dbufs[par], dst_hbm.at[pl.ds(0, 4096)],
                out_sems[par]).wait()

        def do_slab(kk, par):
            f, v0 = slab_of(kk)
            wait_in(par)

            def trow(e, carry):
                esp = jnp.full((16,), e, jnp.int32)
                vals = [sbuf[par, e, pl.ds(kx * 16, 16)] for kx in range(8)]
                idxs = [patt[kx] + esp for kx in range(8)]
                for kx in range(8):
                    plsc.store_scatter(dbufs[par], [idxs[kx]], vals[kx])
                return carry

            lax.fori_loop(0, 32, trow, 0, unroll=2)
            pltpu.async_copy(
                dbufs[par],
                dst_hbm.at[pl.ds(
                    pl.multiple_of((f * (V // 4) + v0 // 4) * 128, 1024),
                    4096)],
                out_sems[par])

        for p in range(NBUF - 1):
            fire_in(p, p)

        def body4(t, carry):
            for off in range(NBUF):
                kk = NBUF * t + off
                nxt = kk + NBUF - 1

                @pl.when((nxt < SLAB_IT) & (w + nxt * NW < NSLAB))
                def _():
                    fire_in(nxt, (off + NBUF - 1) % NBUF)

                @pl.when(w + kk * NW < NSLAB)
                def _():
                    @pl.when(kk >= NBUF)
                    def _():
                        wait_out(off)

                    do_slab(kk, off)
            return carry

        lax.fori_loop(0, (SLAB_IT + NBUF - 1) // NBUF, body4, 0)
        for p in range(NBUF):
            wait_out(p)

        @pl.when(w < F)
        def _():
            f = w
            for e2 in range(4):
                pltpu.sync_copy(
                    src_hbm.at[f, pl.ds(e2 * 8, 8), pl.ds(VT * 128, VTAIL)],
                    tbuf.at[pl.ds(e2 * 8, 8)])

            def trow(e, carry):
                esp = jnp.full((16,), e, jnp.int32)
                for kx in range(2):
                    vals = tbuf[e, pl.ds(kx * 16, 16)]
                    plsc.store_scatter(tdbuf, [patt[kx] + esp], vals)
                return carry

            lax.fori_loop(0, 32, trow, 0)
            pltpu.sync_copy(
                tdbuf,
                dst_hbm.at[pl.ds(
                    pl.multiple_of((f * (V // 4) + VT * 32) * 128, 1024),
                    VTAIL * 32)])

    return k(t_t)


def _sc_gather(idx2d, flat_tab):
    mesh = plsc.VectorSubcoreMesh(core_axis_name="c", subcore_axis_name="s")

    @functools.partial(
        pl.kernel,
        mesh=mesh,
        out_type=jax.ShapeDtypeStruct((ROWS, D), jnp.float32),
        compiler_params=pltpu.CompilerParams(use_tc_tiling_on_sc=False),
        scratch_types=[
            pltpu.VMEM((2, JPC, 128), jnp.int32),
            pltpu.VMEM((2, CHUNK, D), jnp.float32),
            pltpu.SemaphoreType.DMA,
            pltpu.SemaphoreType.DMA,
            pltpu.SemaphoreType.DMA,
            pltpu.SemaphoreType.DMA,
            pltpu.SemaphoreType.DMA,
            pltpu.SemaphoreType.DMA,
        ],
    )
    def k(idx_hbm, tab_hbm, out_hbm, idx_v, rows_v,
          idx_s0, idx_s1, gat_s0, gat_s1, out_s0, out_s1):
        wid = lax.axis_index("s") * NC + lax.axis_index("c")
        irow0 = wid * (ROWS_W // 128)
        orow0 = wid * ROWS_W
        lane = lax.broadcasted_iota(jnp.int32, (16,), 0)

        idx_sems = (idx_s0, idx_s1)
        gat_sems = (gat_s0, gat_s1)
        out_sems = (out_s0, out_s1)

        def start_idx(c):
            b = c & 1
            return pltpu.async_copy(
                idx_hbm.at[pl.ds(irow0 + c * JPC, JPC)], idx_v.at[b],
                idx_sems[b])

        idx_cp = {0: start_idx(0)}
        out_cp = {}
        for c in range(NCHUNK):
            b = c & 1
            if c + 1 < NCHUNK:
                idx_cp[c + 1] = start_idx(c + 1)
            idx_cp[c].wait()

            def body(v, carry):
                j = v // 8
                col = (v % 8) * 16
                pos = c * CHUNK + v * 16 + lane
                off = (pos % F) * V
                idx_v[b, j, pl.ds(col, 16)] = (
                    idx_v[b, j, pl.ds(col, 16)] + off)
                return carry

            lax.fori_loop(0, VPC, body, 0)

            if c >= 2:
                out_cp[c - 2].wait()
            gats = [
                pltpu.async_copy(
                    tab_hbm.at[idx_v.at[b, j]],
                    rows_v.at[b, pl.ds(j * 128, 128)], gat_sems[b])
                for j in range(JPC)
            ]
            for g in gats:
                g.wait()
            out_cp[c] = pltpu.async_copy(
                rows_v.at[b], out_hbm.at[pl.ds(orow0 + c * CHUNK, CHUNK)],
                out_sems[b])
        out_cp[NCHUNK - 2].wait()
        out_cp[NCHUNK - 1].wait()

    return k(idx2d, flat_tab)


def kernel(x_cat, tables):
    idx2d = x_cat.reshape(ROWS // 128, 128)
    t_t = tables.transpose(0, 2, 1)
    tab1d = _sc_transpose(t_t)
    flat_tab = tab1d.reshape(F * V, D)
    out = _sc_gather(idx2d, flat_tab)
    return out.reshape(B, F * D)


# speedup vs baseline: 1.5317x; 1.2558x over previous
import functools

import jax
import jax.numpy as jnp
from jax import lax
from jax.experimental import pallas as pl
from jax.experimental.pallas import tpu as pltpu
from jax.experimental.pallas import tpu_sc as plsc

F = 26
V = 100000
D = 32
B = 16384

NC = 2
NS = 16
NW = NC * NS
ROWS = B * F
ROWS_W = ROWS // NW
CHUNK = 1024
NCHUNK = ROWS_W // CHUNK
JPC = CHUNK // 128
VPC = CHUNK // 16

VT = V // 128
VTAIL = V - VT * 128
NSLAB = F * VT
SLAB_IT = (NSLAB + NW - 1) // NW


NBUF = 4


def _sc_transpose(t_t):
    mesh = plsc.VectorSubcoreMesh(core_axis_name="c", subcore_axis_name="s")

    @functools.partial(
        pl.kernel,
        mesh=mesh,
        out_type=jax.ShapeDtypeStruct((F * V * D,), jnp.float32),
        compiler_params=pltpu.CompilerParams(
            use_tc_tiling_on_sc=True, needs_layout_passes=False),
        scratch_types=[
            pltpu.VMEM((NBUF, 32, 128), jnp.float32),
            pltpu.VMEM((4096,), jnp.float32),
            pltpu.VMEM((4096,), jnp.float32),
            pltpu.VMEM((4096,), jnp.float32),
            pltpu.VMEM((4096,), jnp.float32),
            pltpu.VMEM((32, VTAIL), jnp.float32),
            pltpu.VMEM((VTAIL * 32,), jnp.float32),
            pltpu.SemaphoreType.DMA,
            pltpu.SemaphoreType.DMA,
            pltpu.SemaphoreType.DMA,
            pltpu.SemaphoreType.DMA,
            pltpu.SemaphoreType.DMA,
            pltpu.SemaphoreType.DMA,
            pltpu.SemaphoreType.DMA,
            pltpu.SemaphoreType.DMA,
        ],
    )
    def k(src_hbm, dst_hbm, sbuf, dbuf0, dbuf1, dbuf2, dbuf3, tbuf, tdbuf,
          in_s0, in_s1, in_s2, in_s3, out_s0, out_s1, out_s2, out_s3):
        w = lax.axis_index("s") * NC + lax.axis_index("c")
        lane = lax.broadcasted_iota(jnp.int32, (16,), 0)
        dbufs = (dbuf0, dbuf1, dbuf2, dbuf3)
        in_sems = (in_s0, in_s1, in_s2, in_s3)
        out_sems = (out_s0, out_s1, out_s2, out_s3)
        patt = [
            lax.shift_right_logical(lane, 2) * 128 + (lane & 3) * 32 + kx * 512
            for kx in range(8)
        ]

        def slab_of(kk):
            g = w + kk * NW
            return g // VT, (g % VT) * 128

        def fire_in(kk, par):
            f, v0 = slab_of(kk)
            for e2 in range(4):
                pltpu.async_copy(
                    src_hbm.at[f, pl.ds(e2 * 8, 8), pl.ds(v0, 128)],
                    sbuf.at[par, pl.ds(e2 * 8, 8)], in_sems[par])

        def wait_in(par):
            for _ in range(4):
                pltpu.make_async_copy(
                    src_hbm.at[0, pl.ds(0, 8), pl.ds(0, 128)],
                    sbuf.at[par, pl.ds(0, 8)], in_sems[par]).wait()

        def wait_out(par):
            pltpu.make_async_copy(
                dbufs[par], dst_hbm.at[pl.ds(0, 4096)],
                out_sems[par]).wait()

        def do_slab(kk, par):
            f, v0 = slab_of(kk)
            wait_in(par)

            def trow(r, carry):
                outs = []
                for dv in range(4):
                    vcol = jnp.full((16,), 4 * r + dv, jnp.int32)
                    for h in range(2):
                        outs.append(plsc.load_gather(
                            sbuf.at[par], [lane + h * 16, vcol]))
                for i8 in range(8):
                    dbufs[par][pl.ds(r * 128 + i8 * 16, 16)] = outs[i8]
                return carry

            lax.fori_loop(0, 32, trow, 0, unroll=2)
            pltpu.async_copy(
                dbufs[par],
                dst_hbm.at[pl.ds(
                    pl.multiple_of((f * (V // 4) + v0 // 4) * 128, 1024),
                    4096)],
                out_sems[par])

        for p in range(NBUF - 1):
            fire_in(p, p)

        def body4(t, carry):
            for off in range(NBUF):
                kk = NBUF * t + off
                nxt = kk + NBUF - 1

                @pl.when((nxt < SLAB_IT) & (w + nxt * NW < NSLAB))
                def _():
                    fire_in(nxt, (off + NBUF - 1) % NBUF)

                @pl.when(w + kk * NW < NSLAB)
                def _():
                    @pl.when(kk >= NBUF)
                    def _():
                        wait_out(off)

                    do_slab(kk, off)
            return carry

        lax.fori_loop(0, (SLAB_IT + NBUF - 1) // NBUF, body4, 0)
        for p in range(NBUF):
            wait_out(p)

        @pl.when(w < F)
        def _():
            f = w
            for e2 in range(4):
                pltpu.sync_copy(
                    src_hbm.at[f, pl.ds(e2 * 8, 8), pl.ds(VT * 128, VTAIL)],
                    tbuf.at[pl.ds(e2 * 8, 8)])

            def ttrow(r, carry):
                outs = []
                for dv in range(4):
                    vcol = jnp.full((16,), 4 * r + dv, jnp.int32)
                    for h in range(2):
                        outs.append(plsc.load_gather(
                            tbuf, [lane + h * 16, vcol]))
                for i8 in range(8):
                    tdbuf[pl.ds(r * 128 + i8 * 16, 16)] = outs[i8]
                return carry

            lax.fori_loop(0, VTAIL // 4, ttrow, 0)
            pltpu.sync_copy(
                tdbuf,
                dst_hbm.at[pl.ds(
                    pl.multiple_of((f * (V // 4) + VT * 32) * 128, 1024),
                    VTAIL * 32)])

    return k(t_t)


def _sc_gather(idx2d, flat_tab):
    mesh = plsc.VectorSubcoreMesh(core_axis_name="c", subcore_axis_name="s")

    @functools.partial(
        pl.kernel,
        mesh=mesh,
        out_type=jax.ShapeDtypeStruct((ROWS, D), jnp.float32),
        compiler_params=pltpu.CompilerParams(use_tc_tiling_on_sc=False),
        scratch_types=[
            pltpu.VMEM((2, JPC, 128), jnp.int32),
            pltpu.VMEM((2, CHUNK, D), jnp.float32),
            pltpu.SemaphoreType.DMA,
            pltpu.SemaphoreType.DMA,
            pltpu.SemaphoreType.DMA,
            pltpu.SemaphoreType.DMA,
            pltpu.SemaphoreType.DMA,
            pltpu.SemaphoreType.DMA,
        ],
    )
    def k(idx_hbm, tab_hbm, out_hbm, idx_v, rows_v,
          idx_s0, idx_s1, gat_s0, gat_s1, out_s0, out_s1):
        wid = lax.axis_index("s") * NC + lax.axis_index("c")
        irow0 = wid * (ROWS_W // 128)
        orow0 = wid * ROWS_W
        lane = lax.broadcasted_iota(jnp.int32, (16,), 0)

        idx_sems = (idx_s0, idx_s1)
        gat_sems = (gat_s0, gat_s1)
        out_sems = (out_s0, out_s1)

        def start_idx(c):
            b = c & 1
            return pltpu.async_copy(
                idx_hbm.at[pl.ds(irow0 + c * JPC, JPC)], idx_v.at[b],
                idx_sems[b])

        idx_cp = {0: start_idx(0)}
        out_cp = {}
        for c in range(NCHUNK):
            b = c & 1
            if c + 1 < NCHUNK:
                idx_cp[c + 1] = start_idx(c + 1)
            idx_cp[c].wait()

            def body(v, carry):
                j = v // 8
                col = (v % 8) * 16
                pos = c * CHUNK + v * 16 + lane
                off = (pos % F) * V
                idx_v[b, j, pl.ds(col, 16)] = (
                    idx_v[b, j, pl.ds(col, 16)] + off)
                return carry

            lax.fori_loop(0, VPC, body, 0)

            if c >= 2:
                out_cp[c - 2].wait()
            gats = [
                pltpu.async_copy(
                    tab_hbm.at[idx_v.at[b, j]],
                    rows_v.at[b, pl.ds(j * 128, 128)], gat_sems[b])
                for j in range(JPC)
            ]
            for g in gats:
                g.wait()
            out_cp[c] = pltpu.async_copy(
                rows_v.at[b], out_hbm.at[pl.ds(orow0 + c * CHUNK, CHUNK)],
                out_sems[b])
        out_cp[NCHUNK - 2].wait()
        out_cp[NCHUNK - 1].wait()

    return k(idx2d, flat_tab)


def kernel(x_cat, tables):
    idx2d = x_cat.reshape(ROWS // 128, 128)
    t_t = tables.transpose(0, 2, 1)
    tab1d = _sc_transpose(t_t)
    flat_tab = tab1d.reshape(F * V, D)
    out = _sc_gather(idx2d, flat_tab)
    return out.reshape(B, F * D)


# speedup vs baseline: 5.1592x; 3.3684x over previous
import functools

import jax
import jax.numpy as jnp
from jax import lax
from jax.experimental import pallas as pl
from jax.experimental.pallas import tpu as pltpu
from jax.experimental.pallas import tpu_sc as plsc

F = 26
V = 100000
D = 32
B = 16384

NC = 2
NS = 16
NW = NC * NS
ROWS = B * F
ROWS_W = ROWS // NW
CHUNK = 1024
NCHUNK = ROWS_W // CHUNK
JPC = CHUNK // 128
VPC = CHUNK // 16

VT = V // 128
VTAIL = V - VT * 128
NSLAB = F * VT
SLAB_IT = (NSLAB + NW - 1) // NW


NBUF = 4


def _sc_transpose(t_t):
    mesh = plsc.VectorSubcoreMesh(core_axis_name="c", subcore_axis_name="s")

    @functools.partial(
        pl.kernel,
        mesh=mesh,
        out_type=jax.ShapeDtypeStruct((F * V * D,), jnp.float32),
        compiler_params=pltpu.CompilerParams(
            use_tc_tiling_on_sc=True, needs_layout_passes=False),
        scratch_types=[
            pltpu.VMEM((NBUF, 32, 128), jnp.float32),
            pltpu.VMEM((4096,), jnp.float32),
            pltpu.VMEM((4096,), jnp.float32),
            pltpu.VMEM((4096,), jnp.float32),
            pltpu.VMEM((4096,), jnp.float32),
            pltpu.VMEM((32, VTAIL), jnp.float32),
            pltpu.VMEM((VTAIL * 32,), jnp.float32),
            pltpu.SemaphoreType.DMA,
            pltpu.SemaphoreType.DMA,
            pltpu.SemaphoreType.DMA,
            pltpu.SemaphoreType.DMA,
            pltpu.SemaphoreType.DMA,
            pltpu.SemaphoreType.DMA,
            pltpu.SemaphoreType.DMA,
            pltpu.SemaphoreType.DMA,
        ],
    )
    def k(src_hbm, dst_hbm, sbuf, dbuf0, dbuf1, dbuf2, dbuf3, tbuf, tdbuf,
          in_s0, in_s1, in_s2, in_s3, out_s0, out_s1, out_s2, out_s3):
        w = lax.axis_index("s") * NC + lax.axis_index("c")
        lane = lax.broadcasted_iota(jnp.int32, (16,), 0)
        dbufs = (dbuf0, dbuf1, dbuf2, dbuf3)
        in_sems = (in_s0, in_s1, in_s2, in_s3)
        out_sems = (out_s0, out_s1, out_s2, out_s3)
        patt = [
            lax.shift_right_logical(lane, 2) * 128 + (lane & 3) * 32 + kx * 512
            for kx in range(8)
        ]

        def slab_of(kk):
            g = w + kk * NW
            return g // VT, (g % VT) * 128

        def fire_in(kk, par):
            f, v0 = slab_of(kk)
            for e2 in range(4):
                pltpu.async_copy(
                    src_hbm.at[f, pl.ds(e2 * 8, 8), pl.ds(v0, 128)],
                    sbuf.at[par, pl.ds(e2 * 8, 8)], in_sems[par])

        def wait_in(par):
            for _ in range(4):
                pltpu.make_async_copy(
                    src_hbm.at[0, pl.ds(0, 8), pl.ds(0, 128)],
                    sbuf.at[par, pl.ds(0, 8)], in_sems[par]).wait()

        def wait_out(par):
            pltpu.make_async_copy(
                dbufs[par], dst_hbm.at[pl.ds(0, 4096)],
                out_sems[par]).wait()

        def do_slab(kk, par):
            f, v0 = slab_of(kk)
            wait_in(par)

            @functools.partial(plsc.parallel_loop, 0, 32, unroll=4)
            def _trow(r):
                outs = []
                for dv in range(4):
                    vcol = jnp.full((16,), 4 * r + dv, jnp.int32)
                    for h in range(2):
                        outs.append(plsc.load_gather(
                            sbuf.at[par], [lane + h * 16, vcol]))
                for i8 in range(8):
                    dbufs[par][pl.ds(r * 128 + i8 * 16, 16)] = outs[i8]
            pltpu.async_copy(
                dbufs[par],
                dst_hbm.at[pl.ds(
                    pl.multiple_of((f * (V // 4) + v0 // 4) * 128, 1024),
                    4096)],
                out_sems[par])

        for p in range(NBUF - 1):
            fire_in(p, p)

        def body4(t, carry):
            for off in range(NBUF):
                kk = NBUF * t + off
                nxt = kk + NBUF - 1

                @pl.when((nxt < SLAB_IT) & (w + nxt * NW < NSLAB))
                def _():
                    fire_in(nxt, (off + NBUF - 1) % NBUF)

                @pl.when(w + kk * NW < NSLAB)
                def _():
                    @pl.when(kk >= NBUF)
                    def _():
                        wait_out(off)

                    do_slab(kk, off)
            return carry

        lax.fori_loop(0, (SLAB_IT + NBUF - 1) // NBUF, body4, 0)
        for p in range(NBUF):
            wait_out(p)

        @pl.when(w < F)
        def _():
            f = w
            for e2 in range(4):
                pltpu.sync_copy(
                    src_hbm.at[f, pl.ds(e2 * 8, 8), pl.ds(VT * 128, VTAIL)],
                    tbuf.at[pl.ds(e2 * 8, 8)])

            def ttrow(r, carry):
                outs = []
                for dv in range(4):
                    vcol = jnp.full((16,), 4 * r + dv, jnp.int32)
                    for h in range(2):
                        outs.append(plsc.load_gather(
                            tbuf, [lane + h * 16, vcol]))
                for i8 in range(8):
                    tdbuf[pl.ds(r * 128 + i8 * 16, 16)] = outs[i8]
                return carry

            lax.fori_loop(0, VTAIL // 4, ttrow, 0)
            pltpu.sync_copy(
                tdbuf,
                dst_hbm.at[pl.ds(
                    pl.multiple_of((f * (V // 4) + VT * 32) * 128, 1024),
                    VTAIL * 32)])

    return k(t_t)


def _sc_gather(idx2d, flat_tab):
    mesh = plsc.VectorSubcoreMesh(core_axis_name="c", subcore_axis_name="s")

    @functools.partial(
        pl.kernel,
        mesh=mesh,
        out_type=jax.ShapeDtypeStruct((ROWS, D), jnp.float32),
        compiler_params=pltpu.CompilerParams(use_tc_tiling_on_sc=False),
        scratch_types=[
            pltpu.VMEM((2, JPC, 128), jnp.int32),
            pltpu.VMEM((2, CHUNK, D), jnp.float32),
            pltpu.SemaphoreType.DMA,
            pltpu.SemaphoreType.DMA,
            pltpu.SemaphoreType.DMA,
            pltpu.SemaphoreType.DMA,
            pltpu.SemaphoreType.DMA,
            pltpu.SemaphoreType.DMA,
        ],
    )
    def k(idx_hbm, tab_hbm, out_hbm, idx_v, rows_v,
          idx_s0, idx_s1, gat_s0, gat_s1, out_s0, out_s1):
        wid = lax.axis_index("s") * NC + lax.axis_index("c")
        irow0 = wid * (ROWS_W // 128)
        orow0 = wid * ROWS_W
        lane = lax.broadcasted_iota(jnp.int32, (16,), 0)

        idx_sems = (idx_s0, idx_s1)
        gat_sems = (gat_s0, gat_s1)
        out_sems = (out_s0, out_s1)

        def start_idx(c):
            b = c & 1
            return pltpu.async_copy(
                idx_hbm.at[pl.ds(irow0 + c * JPC, JPC)], idx_v.at[b],
                idx_sems[b])

        idx_cp = {0: start_idx(0)}
        out_cp = {}
        for c in range(NCHUNK):
            b = c & 1
            if c + 1 < NCHUNK:
                idx_cp[c + 1] = start_idx(c + 1)
            idx_cp[c].wait()

            def body(v, carry):
                j = v // 8
                col = (v % 8) * 16
                pos = c * CHUNK + v * 16 + lane
                off = (pos % F) * V
                idx_v[b, j, pl.ds(col, 16)] = (
                    idx_v[b, j, pl.ds(col, 16)] + off)
                return carry

            lax.fori_loop(0, VPC, body, 0)

            if c >= 2:
                out_cp[c - 2].wait()
            gats = [
                pltpu.async_copy(
                    tab_hbm.at[idx_v.at[b, j]],
                    rows_v.at[b, pl.ds(j * 128, 128)], gat_sems[b])
                for j in range(JPC)
            ]
            for g in gats:
                g.wait()
            out_cp[c] = pltpu.async_copy(
                rows_v.at[b], out_hbm.at[pl.ds(orow0 + c * CHUNK, CHUNK)],
                out_sems[b])
        out_cp[NCHUNK - 2].wait()
        out_cp[NCHUNK - 1].wait()

    return k(idx2d, flat_tab)


def kernel(x_cat, tables):
    idx2d = x_cat.reshape(ROWS // 128, 128)
    t_t = tables.transpose(0, 2, 1)
    tab1d = _sc_transpose(t_t)
    flat_tab = tab1d.reshape(F * V, D)
    out = _sc_gather(idx2d, flat_tab)
    return out.reshape(B, F * D)
